# Initial kernel scaffold; baseline (speedup 1.0000x reference)
import jax
import jax.numpy as jnp
from jax.experimental import pallas as pl
from jax.experimental.pallas import tpu as pltpu
from jax.experimental.pallas import tpu_sc as plsc

EMB_DIM = 64
WINDOW = 256


def kernel(x, table):
    batch, hist = x.shape
    n = batch * hist
    assert n % WINDOW == 0
    idx = x.reshape(1, n)

    mesh = plsc.VectorSubcoreMesh(core_axis_name="c", subcore_axis_name="s")

    @pl.kernel(
        out_type=jax.ShapeDtypeStruct((n, EMB_DIM), table.dtype),
        mesh=mesh,
    )
    def gather_kernel(table_hbm, idx_hbm, out_hbm):
        def body(i_vmem, o_vmem):
            pltpu.sync_copy(table_hbm.at[i_vmem.at[0]], o_vmem)

        pltpu.emit_pipeline(
            body,
            grid=(n // WINDOW,),
            in_specs=[pl.BlockSpec((1, WINDOW), index_map=lambda i: (0, i))],
            out_specs=[pl.BlockSpec((WINDOW, EMB_DIM), index_map=lambda i: (i, 0))],
            core_axis_name=("c", "s"),
            dimension_semantics=(pltpu.PARALLEL,),
        )(idx_hbm, out_hbm)

    out = gather_kernel(table, idx)
    return out.reshape(batch, hist, EMB_DIM)


# speedup vs baseline: 1.6032x; 1.6032x over previous
import jax
import jax.numpy as jnp
from jax import lax
from jax.experimental import pallas as pl
from jax.experimental.pallas import tpu as pltpu
from jax.experimental.pallas import tpu_sc as plsc

EMB_DIM = 64
WIDE = 128
NUM_WORKERS = 32
CHUNK = 512


def kernel(x, table):
    batch, hist = x.shape
    n = batch * hist
    per_worker = n // NUM_WORKERS
    assert per_worker * NUM_WORKERS == n and per_worker % CHUNK == 0
    idx = x.reshape(n)
    table_w = jnp.pad(table, ((0, 0), (0, WIDE - EMB_DIM)))

    mesh = plsc.VectorSubcoreMesh(core_axis_name="c", subcore_axis_name="s")

    @pl.kernel(
        out_type=jax.ShapeDtypeStruct((n, WIDE), table.dtype),
        mesh=mesh,
        scratch_types=[
            pltpu.VMEM((CHUNK,), jnp.int32),
            pltpu.VMEM((CHUNK, WIDE), jnp.float32),
            pltpu.SemaphoreType.DMA,
        ],
    )
    def gather_kernel(table_hbm, idx_hbm, out_hbm, idx_v, rows_v, sem):
        wid = lax.axis_index("s") * 2 + lax.axis_index("c")
        base = wid * per_worker

        @pl.loop(0, per_worker, step=CHUNK)
        def _(off):
            pltpu.sync_copy(idx_hbm.at[pl.ds(base + off, CHUNK)], idx_v)
            pltpu.async_copy(table_hbm.at[idx_v], rows_v, sem).wait()
            pltpu.sync_copy(rows_v, out_hbm.at[pl.ds(base + off, CHUNK)])

    out = gather_kernel(table_w, idx)
    return out[:, :EMB_DIM].reshape(batch, hist, EMB_DIM)


# speedup vs baseline: 1.7957x; 1.1201x over previous
import dataclasses

import jax
import jax.numpy as jnp
from jax import lax
from jax.experimental import pallas as pl
from jax.experimental.pallas import tpu as pltpu
from jax.experimental.pallas import tpu_sc as plsc

EMB_DIM = 64
NUM_WORKERS = 32
CHUNK = 512


def kernel(x, table):
    batch, hist = x.shape
    n = batch * hist
    per_worker = n // NUM_WORKERS
    assert per_worker * NUM_WORKERS == n and per_worker % CHUNK == 0
    idx = x.reshape(n)

    mesh = plsc.VectorSubcoreMesh(core_axis_name="c", subcore_axis_name="s")
    cp = dataclasses.replace(pltpu.CompilerParams(), use_tc_tiling_on_sc=False)

    @pl.kernel(
        out_type=jax.ShapeDtypeStruct((n, EMB_DIM), table.dtype),
        mesh=mesh,
        scratch_types=[
            pltpu.VMEM((CHUNK,), jnp.int32),
            pltpu.VMEM((CHUNK, EMB_DIM), jnp.float32),
            pltpu.SemaphoreType.DMA,
        ],
        compiler_params=cp,
    )
    def gather_kernel(table_hbm, idx_hbm, out_hbm, idx_v, rows_v, sem):
        wid = lax.axis_index("s") * 2 + lax.axis_index("c")
        base = wid * per_worker

        @pl.loop(0, per_worker, step=CHUNK)
        def _(off):
            pltpu.sync_copy(idx_hbm.at[pl.ds(base + off, CHUNK)], idx_v)
            pltpu.async_copy(table_hbm.at[idx_v], rows_v, sem).wait()
            pltpu.sync_copy(rows_v, out_hbm.at[pl.ds(base + off, CHUNK)])

    out = gather_kernel(table, idx)
    return out.reshape(batch, hist, EMB_DIM)


# speedup vs baseline: 1.8644x; 1.0383x over previous
import dataclasses

import jax
import jax.numpy as jnp
from jax import lax
from jax.experimental import pallas as pl
from jax.experimental.pallas import tpu as pltpu
from jax.experimental.pallas import tpu_sc as plsc

EMB_DIM = 64
NUM_WORKERS = 32
NB = 8


def kernel(x, table):
    batch, hist = x.shape
    per_worker = batch // NUM_WORKERS
    n_chunks = per_worker // NB
    assert per_worker * NUM_WORKERS == batch and n_chunks * NB == per_worker
    assert n_chunks % 2 == 0

    mesh = plsc.VectorSubcoreMesh(core_axis_name="c", subcore_axis_name="s")
    cp = dataclasses.replace(pltpu.CompilerParams(), use_tc_tiling_on_sc=False)

    @pl.kernel(
        out_type=jax.ShapeDtypeStruct((batch, hist, EMB_DIM), table.dtype),
        mesh=mesh,
        scratch_types=[
            pltpu.VMEM((NB, hist), jnp.int32),
            pltpu.VMEM((NB, hist), jnp.int32),
            pltpu.VMEM((NB, hist, EMB_DIM), jnp.float32),
            pltpu.VMEM((NB, hist, EMB_DIM), jnp.float32),
            pltpu.SemaphoreType.DMA,
            pltpu.SemaphoreType.DMA,
            pltpu.SemaphoreType.DMA,
            pltpu.SemaphoreType.DMA,
        ],
        compiler_params=cp,
    )
    def gather_kernel(
        x_hbm, table_hbm, out_hbm, idx0, idx1, sv0, sv1, semg0, semg1, semo0, semo1
    ):
        wid = lax.axis_index("s") * 2 + lax.axis_index("c")
        b0 = wid * per_worker

        def fire(chunk, idx_v, s_v, semg):
            pltpu.sync_copy(x_hbm.at[pl.ds(b0 + chunk * NB, NB)], idx_v)
            for j in range(NB):
                pltpu.async_copy(table_hbm.at[idx_v.at[j]], s_v.at[j], semg)

        def drain(idx_v, s_v, semg):
            for j in range(NB):
                pltpu.make_async_copy(table_hbm.at[idx_v.at[j]], s_v.at[j], semg).wait()

        def store(chunk, s_v, semo):
            return pltpu.async_copy(
                s_v, out_hbm.at[pl.ds(b0 + chunk * NB, NB)], semo
            )

        def store_wait(chunk, s_v, semo):
            pltpu.make_async_copy(
                s_v, out_hbm.at[pl.ds(b0 + chunk * NB, NB)], semo
            ).wait()

        fire(0, idx0, sv0, semg0)
        fire(1, idx1, sv1, semg1)

        @pl.loop(0, n_chunks // 2 - 1)
        def _(i):
            ca = 2 * i
            cb = ca + 1
            drain(idx0, sv0, semg0)
            store(ca, sv0, semo0)
            drain(idx1, sv1, semg1)
            store(cb, sv1, semo1)
            store_wait(ca, sv0, semo0)
            fire(ca + 2, idx0, sv0, semg0)
            store_wait(cb, sv1, semo1)
            fire(cb + 2, idx1, sv1, semg1)

        drain(idx0, sv0, semg0)
        store(n_chunks - 2, sv0, semo0)
        drain(idx1, sv1, semg1)
        store(n_chunks - 1, sv1, semo1)
        store_wait(n_chunks - 2, sv0, semo0)
        store_wait(n_chunks - 1, sv1, semo1)

    return gather_kernel(x, table)
